# baseline (speedup 1.0000x reference)
import functools

import jax
import jax.numpy as jnp
from jax import lax
from jax.experimental import pallas as pl
from jax.experimental.pallas import tpu as pltpu
from jax.experimental.pallas import tpu_sc as plsc

_NUM_CORES = 2
_NUM_SUBCORES = 16
_NUM_WORKERS = _NUM_CORES * _NUM_SUBCORES
_LANES = 16
_CHUNK = 64


def _relayout_sc(X):
    V, D = X.shape
    R = 128 // D
    n_blk = V // R
    rows_per_chunk = _CHUNK * R
    n_chunks = -(-V // rows_per_chunk)
    max_per_w = -(-n_chunks // _NUM_WORKERS)
    mesh = plsc.VectorSubcoreMesh(core_axis_name="c", subcore_axis_name="s")

    @functools.partial(
        pl.kernel,
        mesh=mesh,
        out_type=jax.ShapeDtypeStruct((n_blk, 128), X.dtype),
        scratch_types=[
            pltpu.VMEM((rows_per_chunk, D), X.dtype),
            pltpu.VMEM((_CHUNK, 128), X.dtype),
            pltpu.SemaphoreType.DMA,
        ],
    )
    def k(table_hbm, dense_hbm, buf_p, buf_d, sem):
        wid = lax.axis_index("s") * _NUM_CORES + lax.axis_index("c")

        @pl.loop(0, max_per_w)
        def _(t):
            cid = wid + t * _NUM_WORKERS

            @pl.when(cid < n_chunks)
            def _():
                blk0 = jnp.minimum(cid * _CHUNK, n_blk - _CHUNK)
                pltpu.sync_copy(
                    table_hbm.at[pl.ds(blk0 * R, rows_per_chunk)], buf_p
                )

                @pl.loop(0, _CHUNK)
                def _(g):
                    for s in range(R):
                        buf_d[g, pl.ds(s * D, D)] = buf_p[g * R + s, :]

                pltpu.sync_copy(buf_d, dense_hbm.at[pl.ds(blk0, _CHUNK)])

    return k(X)


def _gather_sc(i, Xd, B, D):
    b_per_w = B // _NUM_WORKERS
    n_dma = b_per_w // 128
    n_grp = b_per_w // _LANES
    mesh = plsc.VectorSubcoreMesh(core_axis_name="c", subcore_axis_name="s")

    @functools.partial(
        pl.kernel,
        mesh=mesh,
        out_type=jax.ShapeDtypeStruct((B * D,), Xd.dtype),
        compiler_params=pltpu.CompilerParams(needs_layout_passes=False),
        scratch_types=[
            pltpu.VMEM((b_per_w,), jnp.int32),
            pltpu.VMEM((n_dma, 128), jnp.int32),
            pltpu.VMEM((b_per_w, 128), Xd.dtype),
            pltpu.VMEM((b_per_w * D,), Xd.dtype),
            pltpu.SemaphoreType.DMA,
        ],
    )
    def k(table_hbm, idx_hbm, out_hbm, idx_v, blk_v, rows_v, out_v, sem):
        wid = lax.axis_index("s") * _NUM_CORES + lax.axis_index("c")
        base = wid * b_per_w
        pltpu.sync_copy(idx_hbm.at[pl.ds(base, b_per_w)], idx_v)

        @pl.loop(0, n_dma)
        def _(c):
            for w in range(128 // _LANES):
                v = idx_v[pl.ds(c * 128 + w * _LANES, _LANES)]
                blk_v[c, pl.ds(w * _LANES, _LANES)] = v >> 3

        copies = [
            pltpu.async_copy(
                table_hbm.at[blk_v.at[c]],
                rows_v.at[pl.ds(c * 128, 128)],
                sem,
            )
            for c in range(n_dma)
        ]
        for c in copies:
            c.wait()

        lane = lax.iota(jnp.int32, _LANES)

        @pl.loop(0, n_grp)
        def _(u):
            v = idx_v[pl.ds(u * _LANES, _LANES)]
            colb = (v & 7) * D
            r = u * _LANES + lane
            o0 = r * D
            for kk in range(D):
                val = plsc.load_gather(rows_v, [r, colb + kk])
                plsc.store_scatter(out_v, [o0 + kk], val)

        pltpu.sync_copy(out_v, out_hbm.at[pl.ds(base * D, b_per_w * D)])

    return k(Xd, i)


def kernel(i, X):
    B = i.shape[0]
    V, D = X.shape
    Xd = _relayout_sc(X)
    out = _gather_sc(i.astype(jnp.int32), Xd, B, D)
    return out.reshape(B, D)


# speedup vs baseline: 1.0049x; 1.0049x over previous
import functools

import jax
import jax.numpy as jnp
from jax import lax
from jax.experimental import pallas as pl
from jax.experimental.pallas import tpu as pltpu
from jax.experimental.pallas import tpu_sc as plsc

_NUM_CORES = 2
_NUM_SUBCORES = 16
_NUM_WORKERS = _NUM_CORES * _NUM_SUBCORES
_LANES = 16
_CHUNK = 64


def _relayout_sc(X):
    V, D = X.shape
    R = 128 // D
    n_blk = V // R
    rows_per_chunk = _CHUNK * R
    n_chunks = -(-V // rows_per_chunk)
    max_per_w = -(-n_chunks // _NUM_WORKERS)
    mesh = plsc.VectorSubcoreMesh(core_axis_name="c", subcore_axis_name="s")

    @functools.partial(
        pl.kernel,
        mesh=mesh,
        out_type=jax.ShapeDtypeStruct((n_blk, 128), X.dtype),
        scratch_types=[
            pltpu.VMEM((rows_per_chunk, D), X.dtype),
            pltpu.VMEM((_CHUNK, 128), X.dtype),
            pltpu.SemaphoreType.DMA,
        ],
    )
    def k(table_hbm, dense_hbm, buf_p, buf_d, sem):
        wid = lax.axis_index("s") * _NUM_CORES + lax.axis_index("c")

        @pl.loop(0, max_per_w)
        def _(t):
            cid = wid + t * _NUM_WORKERS

            @pl.when(cid < n_chunks)
            def _():
                blk0 = jnp.minimum(cid * _CHUNK, n_blk - _CHUNK)
                pltpu.sync_copy(
                    table_hbm.at[pl.ds(blk0 * R, rows_per_chunk)], buf_p
                )

                @pl.loop(0, _CHUNK // 8)
                def _(g8):
                    for gg in range(8):
                        g = g8 * 8 + gg
                        for s in range(R):
                            buf_d[g, pl.ds(s * D, D)] = buf_p[g * R + s, :]

                pltpu.sync_copy(buf_d, dense_hbm.at[pl.ds(blk0, _CHUNK)])

    return k(X)


def _gather_sc(i, Xd, B, D):
    b_per_w = B // _NUM_WORKERS
    n_dma = b_per_w // 128
    n_grp = b_per_w // _LANES
    mesh = plsc.VectorSubcoreMesh(core_axis_name="c", subcore_axis_name="s")

    @functools.partial(
        pl.kernel,
        mesh=mesh,
        out_type=jax.ShapeDtypeStruct((B * D,), Xd.dtype),
        compiler_params=pltpu.CompilerParams(needs_layout_passes=False),
        scratch_types=[
            pltpu.VMEM((b_per_w,), jnp.int32),
            pltpu.VMEM((n_dma, 128), jnp.int32),
            pltpu.VMEM((b_per_w, 128), Xd.dtype),
            pltpu.VMEM((b_per_w * D,), Xd.dtype),
            pltpu.SemaphoreType.DMA,
        ],
    )
    def k(table_hbm, idx_hbm, out_hbm, idx_v, blk_v, rows_v, out_v, sem):
        wid = lax.axis_index("s") * _NUM_CORES + lax.axis_index("c")
        base = wid * b_per_w
        pltpu.sync_copy(idx_hbm.at[pl.ds(base, b_per_w)], idx_v)

        @pl.loop(0, n_dma)
        def _(c):
            for w in range(128 // _LANES):
                v = idx_v[pl.ds(c * 128 + w * _LANES, _LANES)]
                blk_v[c, pl.ds(w * _LANES, _LANES)] = v >> 3

        copies = [
            pltpu.async_copy(
                table_hbm.at[blk_v.at[c]],
                rows_v.at[pl.ds(c * 128, 128)],
                sem,
            )
            for c in range(n_dma)
        ]
        for c in copies:
            c.wait()

        lane = lax.iota(jnp.int32, _LANES)

        @pl.loop(0, n_grp)
        def _(u):
            v = idx_v[pl.ds(u * _LANES, _LANES)]
            colb = (v & 7) * D
            r = u * _LANES + lane
            o0 = r * D
            for kk in range(D):
                val = plsc.load_gather(rows_v, [r, colb + kk])
                plsc.store_scatter(out_v, [o0 + kk], val)

        pltpu.sync_copy(out_v, out_hbm.at[pl.ds(base * D, b_per_w * D)])

    return k(Xd, i)


def kernel(i, X):
    B = i.shape[0]
    V, D = X.shape
    Xd = _relayout_sc(X)
    out = _gather_sc(i.astype(jnp.int32), Xd, B, D)
    return out.reshape(B, D)


# speedup vs baseline: 1.3367x; 1.3302x over previous
import functools

import jax
import jax.numpy as jnp
from jax import lax
from jax.experimental import pallas as pl
from jax.experimental.pallas import tpu as pltpu
from jax.experimental.pallas import tpu_sc as plsc

_NUM_CORES = 2
_NUM_SUBCORES = 16
_NUM_WORKERS = _NUM_CORES * _NUM_SUBCORES


def _gather_sc(i, X):
    (B,) = i.shape
    V, D = X.shape
    b_per_w = B // _NUM_WORKERS
    n_dma = b_per_w // 128
    mesh = plsc.VectorSubcoreMesh(core_axis_name="c", subcore_axis_name="s")

    @functools.partial(
        pl.kernel,
        mesh=mesh,
        out_type=jax.ShapeDtypeStruct((B, D), X.dtype),
        compiler_params=pltpu.CompilerParams(use_tc_tiling_on_sc=False),
        scratch_types=[
            pltpu.VMEM((n_dma, 128), jnp.int32),
            pltpu.VMEM((b_per_w, D), X.dtype),
            pltpu.SemaphoreType.DMA,
        ],
    )
    def k(table_hbm, idx_hbm, out_hbm, idx_v, rows_v, sem):
        wid = lax.axis_index("s") * _NUM_CORES + lax.axis_index("c")
        base = wid * b_per_w
        for c in range(n_dma):
            pltpu.sync_copy(
                idx_hbm.at[pl.ds(base + c * 128, 128)], idx_v.at[c]
            )
        copies = [
            pltpu.async_copy(
                table_hbm.at[idx_v.at[c]],
                rows_v.at[pl.ds(c * 128, 128)],
                sem,
            )
            for c in range(n_dma)
        ]
        for c in copies:
            c.wait()
        pltpu.sync_copy(rows_v, out_hbm.at[pl.ds(base, b_per_w)])

    return k(X, i)


def kernel(i, X):
    return _gather_sc(i.astype(jnp.int32), X)


# speedup vs baseline: 1.3583x; 1.0162x over previous
import functools

import jax
import jax.numpy as jnp
from jax import lax
from jax.experimental import pallas as pl
from jax.experimental.pallas import tpu as pltpu
from jax.experimental.pallas import tpu_sc as plsc

_NUM_CORES = 2
_NUM_SUBCORES = 16
_NUM_WORKERS = _NUM_CORES * _NUM_SUBCORES
_LANES = 16
_TC_BLOCK = 1000


def _relayout_tc(X):
    V, D = X.shape
    R = 128 // D
    n_blk = V // R
    grid = n_blk // _TC_BLOCK
    X3 = X.reshape(n_blk, R, D)

    def body(x_ref, o_ref):
        for k in range(R):
            o_ref[:, k * D:(k + 1) * D] = x_ref[:, k, :]

    return pl.pallas_call(
        body,
        grid=(grid,),
        in_specs=[pl.BlockSpec((_TC_BLOCK, R, D), lambda g: (g, 0, 0))],
        out_specs=pl.BlockSpec((_TC_BLOCK, 128), lambda g: (g, 0)),
        out_shape=jax.ShapeDtypeStruct((n_blk, 128), X.dtype),
        compiler_params=pltpu.CompilerParams(
            dimension_semantics=("parallel",)
        ),
    )(X3)


def _gather_sc(i, Xd, B, D):
    b_per_w = B // _NUM_WORKERS
    n_dma = b_per_w // 128
    n_grp = b_per_w // _LANES
    mesh = plsc.VectorSubcoreMesh(core_axis_name="c", subcore_axis_name="s")

    @functools.partial(
        pl.kernel,
        mesh=mesh,
        out_type=jax.ShapeDtypeStruct((B * D,), Xd.dtype),
        compiler_params=pltpu.CompilerParams(needs_layout_passes=False),
        scratch_types=[
            pltpu.VMEM((b_per_w,), jnp.int32),
            pltpu.VMEM((n_dma, 128), jnp.int32),
            pltpu.VMEM((b_per_w, 128), Xd.dtype),
            pltpu.VMEM((b_per_w * D,), Xd.dtype),
            pltpu.SemaphoreType.DMA,
        ],
    )
    def k(table_hbm, idx_hbm, out_hbm, idx_v, blk_v, rows_v, out_v, sem):
        wid = lax.axis_index("s") * _NUM_CORES + lax.axis_index("c")
        base = wid * b_per_w
        pltpu.sync_copy(idx_hbm.at[pl.ds(base, b_per_w)], idx_v)

        @pl.loop(0, n_dma)
        def _(c):
            for w in range(128 // _LANES):
                v = idx_v[pl.ds(c * 128 + w * _LANES, _LANES)]
                blk_v[c, pl.ds(w * _LANES, _LANES)] = v >> 3

        copies = [
            pltpu.async_copy(
                table_hbm.at[blk_v.at[c]],
                rows_v.at[pl.ds(c * 128, 128)],
                sem,
            )
            for c in range(n_dma)
        ]
        for c in copies:
            c.wait()

        lane = lax.iota(jnp.int32, _LANES)

        @pl.loop(0, n_grp)
        def _(u):
            v = idx_v[pl.ds(u * _LANES, _LANES)]
            colb = (v & 7) * D
            r = u * _LANES + lane
            o0 = r * D
            for kk in range(D):
                val = plsc.load_gather(rows_v, [r, colb + kk])
                plsc.store_scatter(out_v, [o0 + kk], val)

        pltpu.sync_copy(out_v, out_hbm.at[pl.ds(base * D, b_per_w * D)])

    return k(Xd, i)


def kernel(i, X):
    B = i.shape[0]
    V, D = X.shape
    Xd = _relayout_tc(X)
    out = _gather_sc(i.astype(jnp.int32), Xd, B, D)
    return out.reshape(B, D)
